# Initial kernel scaffold; baseline (speedup 1.0000x reference)
import jax
import jax.numpy as jnp
from jax.experimental import pallas as pl


def kernel(x, edge_index, W1l, b1l, W1r, W2l, b2l, W2r):
    raise NotImplementedError("write your pallas kernel here")


# speedup vs baseline: 4.8975x; 4.8975x over previous
import functools

import jax
import jax.numpy as jnp
from jax import lax
from jax.experimental import pallas as pl
from jax.experimental.pallas import tpu as pltpu
from jax.experimental.pallas import tpu_sc as plsc

N = 10000
NP = 10240
D = 128
E = 320000

NC = 2
NS = 16
NW = NC * NS
EPT = E // NW
CH = 80
NCHUNK = EPT // CH

NRH = 1
NPH = NP // NRH
TRASH = 512 if NRH > 1 else 0
ACCR = NPH + TRASH
ZB = 64
NZB = ACCR // NS // ZB
NSB = NPH // NS // ZB


def _sc_agg(x, src, dst, with_deg):
    mesh = plsc.VectorSubcoreMesh(core_axis_name="c", subcore_axis_name="s")
    out_type = [jax.ShapeDtypeStruct((NC, NP, D), jnp.float32)]
    if with_deg:
        out_type.append(jax.ShapeDtypeStruct((NC, NP, D), jnp.float32))

    @functools.partial(
        pl.kernel,
        out_type=tuple(out_type),
        mesh=mesh,
        scratch_types=[
            pltpu.VMEM((CH,), jnp.int32),
            pltpu.VMEM((CH,), jnp.int32),
            pltpu.VMEM((CH,), jnp.int32),
            pltpu.VMEM((CH, D), jnp.float32),
            pltpu.VMEM((CH, D), jnp.float32),
            pltpu.VMEM((ZB, D), jnp.float32),
            pltpu.VMEM_SHARED((ACCR, D), jnp.float32),
            pltpu.SemaphoreType.DMA,
        ],
    )
    def k(x_hbm, src_hbm, dst_hbm, *rest):
        if with_deg:
            p_hbm, d_hbm = rest[0], rest[1]
            rest = rest[2:]
        else:
            p_hbm, d_hbm = rest[0], None
            rest = rest[1:]
        srcv, dstv, dstl, rows, ones, stg, acc, sem = rest
        c = lax.axis_index("c")
        s = lax.axis_index("s")
        wid = s * NC + c
        ebase = wid * EPT
        ones16 = jnp.ones((16,), jnp.float32)
        zeros16 = jnp.zeros((16,), jnp.float32)

        def fill_ones(i, _):
            for l in range(D // 16):
                ones[i, pl.ds(16 * l, 16)] = ones16
            return 0
        if with_deg:
            lax.fori_loop(0, CH, fill_ones, 0)

        passes = [("feat", r) for r in range(NRH)]
        if with_deg:
            passes += [("deg", r) for r in range(NRH)]

        for kind, rh in passes:
            feat = kind == "feat"
            out_hbm = p_hbm if feat else d_hbm
            nbase = rh * NPH

            def fill_z(i, _):
                for l in range(D // 16):
                    stg[i, pl.ds(16 * l, 16)] = zeros16
                return 0
            lax.fori_loop(0, ZB, fill_z, 0)
            for b in range(NZB):
                pltpu.sync_copy(stg, acc.at[pl.ds(s * (NZB * ZB) + b * ZB, ZB)])
            plsc.subcore_barrier()

            def edge_body(j, _):
                off = ebase + j * CH
                pltpu.sync_copy(dst_hbm.at[pl.ds(off, CH)], dstv)
                if NRH > 1:
                    for g in range(CH // 16):
                        v = dstv[pl.ds(16 * g, 16)]
                        lo = v - nbase
                        ok = (lo >= 0) & (lo < NPH)
                        trash = NPH + lax.iota(jnp.int32, 16) * 16 + g
                        dstl[pl.ds(16 * g, 16)] = jnp.where(ok, lo, trash)
                    idx = dstl
                else:
                    idx = dstv
                if feat:
                    pltpu.sync_copy(src_hbm.at[pl.ds(off, CH)], srcv)
                    pltpu.async_copy(x_hbm.at[srcv], rows, sem).wait()
                    pltpu.sync_copy(rows, acc.at[idx], add=True)
                else:
                    pltpu.sync_copy(ones, acc.at[idx], add=True)
                return 0
            lax.fori_loop(0, NCHUNK, edge_body, 0)
            plsc.subcore_barrier()

            for b in range(NSB):
                r = s * (NSB * ZB) + b * ZB
                pltpu.sync_copy(acc.at[pl.ds(r, ZB)], stg)
                pltpu.sync_copy(stg, out_hbm.at[c, pl.ds(nbase + r, ZB)])
            plsc.subcore_barrier()

    out = k(x, src, dst)
    return out if isinstance(out, (tuple, list)) else (out,)


def _tc_layer(p, dp, x, Wl, b, Wr, last):
    B = 2000
    grid = N // B

    def body(p_ref, dp_ref, x_ref, wl_ref, b_ref, wr_ref, o_ref):
        deg = jnp.maximum(dp_ref[0, :, 0:1] + dp_ref[1, :, 0:1], 1.0)
        m = (p_ref[0] + p_ref[1]) / deg
        y = (jnp.dot(m, wl_ref[...], preferred_element_type=jnp.float32)
             + b_ref[...]
             + jnp.dot(x_ref[...], wr_ref[...],
                       preferred_element_type=jnp.float32))
        if last:
            mx = jnp.max(y, axis=-1, keepdims=True)
            ex = jnp.exp(y - mx)
            o_ref[...] = y - mx - jnp.log(jnp.sum(ex, axis=-1, keepdims=True))
        else:
            o_ref[...] = jnp.maximum(y, 0.0)

    return pl.pallas_call(
        body,
        grid=(grid,),
        in_specs=[
            pl.BlockSpec((NC, B, D), lambda i: (0, i, 0)),
            pl.BlockSpec((NC, B, D), lambda i: (0, i, 0)),
            pl.BlockSpec((B, D), lambda i: (i, 0)),
            pl.BlockSpec((D, D), lambda i: (0, 0)),
            pl.BlockSpec((1, D), lambda i: (0, 0)),
            pl.BlockSpec((D, D), lambda i: (0, 0)),
        ],
        out_specs=pl.BlockSpec((B, D), lambda i: (i, 0)),
        out_shape=jax.ShapeDtypeStruct((N, D), jnp.float32),
    )(p, dp, x, Wl, b, Wr)


def kernel(x, edge_index, W1l, b1l, W1r, W2l, b2l, W2r):
    src = edge_index[0].astype(jnp.int32)
    dst = edge_index[1].astype(jnp.int32)
    p1, dp = _sc_agg(x, src, dst, with_deg=True)
    h = _tc_layer(p1, dp, x, W1l, b1l.reshape(1, D), W1r, last=False)
    (p2,) = _sc_agg(h, src, dst, with_deg=False)
    return _tc_layer(p2, dp, h, W2l, b2l.reshape(1, D), W2r, last=True)
